# baseline (speedup 1.0000x reference)
import functools

import jax
import jax.numpy as jnp
from jax import lax
from jax.experimental import pallas as pl
from jax.experimental.pallas import tpu as pltpu
from jax.experimental.pallas import tpu_sc as plsc

_BATCH = 16384
_D = 3200
_NUM_GROUPS = 10
_NUM_SLICES = 5
_CHUNK = 32
_GROUP_W = _NUM_SLICES * _CHUNK
_READ_W = 1664

_info = plsc.get_sparse_core_info()
_NC = _info.num_cores
_NS = _info.num_subcores
_NW = _NC * _NS
_RPW = _BATCH // _NW
_CR = 8
_NCHUNK = _RPW // _CR


def _body(in_hbm, *rest, groups):
    ng = len(groups)
    outs = rest[:ng]
    in_bufs = rest[ng : ng + 2]
    out_bufs = [rest[ng + 2 + u * ng :][:ng] for u in (0, 1)]
    sems = rest[ng + 2 + 2 * ng :]
    rsems = sems[0:2]
    wsems = sems[2:4]
    wid = lax.axis_index("s") * _NC + lax.axis_index("c")
    row0 = wid * _RPW

    def read_desc(c, u):
        return pltpu.make_async_copy(
            in_hbm.at[pl.ds(row0 + c * _CR, _CR), pl.ds(0, _READ_W)],
            in_bufs[u],
            rsems[u],
        )

    def write_desc(c, u, gg):
        return pltpu.make_async_copy(
            out_bufs[u][gg],
            outs[gg].at[pl.ds(row0 + c * _CR, _CR), :],
            wsems[u],
        )

    read_desc(0, 0).start()
    read_desc(1, 1).start()

    def chunk_pair(c2, _):
        for u in (0, 1):
            c = c2 * 2 + u
            read_desc(c, u).wait()

            @pl.when(c >= 2)
            def _():
                for gg in range(ng):
                    write_desc(c - 2, u, gg).wait()

            @plsc.parallel_loop(0, _CR, step=1, unroll=2)
            def _(r):
                for gg, g in enumerate(groups):
                    for j in range(_NUM_SLICES):
                        src = (j * _NUM_GROUPS + g) * _CHUNK
                        dst = j * _CHUNK
                        for k in (0, 16):
                            out_bufs[u][gg][r, pl.ds(dst + k, 16)] = in_bufs[
                                u
                            ][r, pl.ds(src + k, 16)]

            for gg in range(ng):
                write_desc(c, u, gg).start()

            @pl.when(c + 2 < _NCHUNK)
            def _():
                read_desc(c + 2, u).start()

        return 0

    lax.fori_loop(0, _NCHUNK // 2, chunk_pair, 0)

    for u in (0, 1):
        for gg in range(ng):
            write_desc(_NCHUNK - 2 + u, u, gg).wait()


def _make_call(groups):
    ng = len(groups)
    out_type = [
        jax.ShapeDtypeStruct((_BATCH, _GROUP_W), jnp.float32)
    ] * ng
    return pl.kernel(
        functools.partial(_body, groups=groups),
        out_type=out_type,
        mesh=plsc.VectorSubcoreMesh(core_axis_name="c", subcore_axis_name="s"),
        scratch_types=(
            [pltpu.VMEM((_CR, _READ_W), jnp.float32)] * 2
            + [pltpu.VMEM((_CR, _GROUP_W), jnp.float32)] * (2 * ng)
            + [pltpu.SemaphoreType.DMA] * 4
        ),
        compiler_params=pltpu.CompilerParams(use_tc_tiling_on_sc=True),
    )


def kernel(input_tensor):
    f1 = _make_call(tuple(range(5)))
    f2 = _make_call(tuple(range(5, 10)))
    return tuple(f1(input_tensor)) + tuple(f2(input_tensor))


# speedup vs baseline: 1.0437x; 1.0437x over previous
import jax
import jax.numpy as jnp
from jax import lax
from jax.experimental import pallas as pl
from jax.experimental.pallas import tpu as pltpu
from jax.experimental.pallas import tpu_sc as plsc

_BATCH = 16384
_D = 3200
_NUM_GROUPS = 10
_NUM_SLICES = 5
_CHUNK = 32
_GROUP_W = _NUM_SLICES * _CHUNK
_READ_W = 1664

_info = plsc.get_sparse_core_info()
_NC = _info.num_cores
_NS = _info.num_subcores
_NW = _NC * _NS
_RPW = _BATCH // _NW
_CR = 8
_NCHUNK = _RPW // _CR


def _body(in_hbm, *rest):
    outs = rest[:_NUM_GROUPS]
    in_bufs = rest[_NUM_GROUPS : _NUM_GROUPS + 2]
    out_bufs = [
        rest[_NUM_GROUPS + 2 + u * _NUM_GROUPS :][:_NUM_GROUPS]
        for u in (0, 1)
    ]
    sems = rest[_NUM_GROUPS + 2 + 2 * _NUM_GROUPS :]
    rsems = sems[0:2]
    wsems = sems[2:4]
    wid = lax.axis_index("s") * _NC + lax.axis_index("c")
    row0 = wid * _RPW

    def read_desc(c, u):
        return pltpu.make_async_copy(
            in_hbm.at[pl.ds(row0 + c * _CR, _CR), pl.ds(0, _READ_W)],
            in_bufs[u],
            rsems[u],
        )

    def write_desc(c, u, g):
        return pltpu.make_async_copy(
            out_bufs[u][g],
            outs[g].at[pl.ds(row0 + c * _CR, _CR), :],
            wsems[u],
        )

    read_desc(0, 0).start()
    read_desc(1, 1).start()

    def chunk_pair(c2, _):
        for u in (0, 1):
            c = c2 * 2 + u
            read_desc(c, u).wait()

            @pl.when(c >= 2)
            def _():
                for g in range(_NUM_GROUPS):
                    write_desc(c - 2, u, g).wait()

            @plsc.parallel_loop(0, _CR, step=1, unroll=2)
            def _(r):
                for g in range(_NUM_GROUPS):
                    for j in range(_NUM_SLICES):
                        src = (j * _NUM_GROUPS + g) * _CHUNK
                        dst = j * _CHUNK
                        for k in (0, 16):
                            out_bufs[u][g][r, pl.ds(dst + k, 16)] = in_bufs[
                                u
                            ][r, pl.ds(src + k, 16)]

            for g in range(_NUM_GROUPS):
                write_desc(c, u, g).start()

            @pl.when(c + 2 < _NCHUNK)
            def _():
                read_desc(c + 2, u).start()

        return 0

    lax.fori_loop(0, _NCHUNK // 2, chunk_pair, 0)

    for u in (0, 1):
        for g in range(_NUM_GROUPS):
            write_desc(_NCHUNK - 2 + u, u, g).wait()


def kernel(input_tensor):
    out_type = [
        jax.ShapeDtypeStruct((_BATCH, _GROUP_W), jnp.float32)
    ] * _NUM_GROUPS
    f = pl.kernel(
        _body,
        out_type=out_type,
        mesh=plsc.VectorSubcoreMesh(core_axis_name="c", subcore_axis_name="s"),
        scratch_types=(
            [pltpu.VMEM((_CR, _READ_W), jnp.float32)] * 2
            + [pltpu.VMEM((_CR, _GROUP_W), jnp.float32)] * (2 * _NUM_GROUPS)
            + [pltpu.SemaphoreType.DMA] * 4
        ),
        compiler_params=pltpu.CompilerParams(use_tc_tiling_on_sc=True),
    )
    return tuple(f(input_tensor))
